# Initial kernel scaffold; baseline (speedup 1.0000x reference)
import jax
import jax.numpy as jnp
from jax.experimental import pallas as pl


def kernel(x, W1, b1, W2, b2, W3, b3, Wg1, bg1, Wg2, bg2, temperature):
    raise NotImplementedError("write your pallas kernel here")


# speedup vs baseline: 3.2355x; 3.2355x over previous
import functools

import jax
import jax.numpy as jnp
from jax.experimental import pallas as pl
from jax.experimental.pallas import tpu as pltpu

IN_DIM = 2048
HID = 1024
E = 8
TOPK = 2
NC = 2
GATE_H = 256
TOKENS = 4096
BT = 512
TB = TOKENS // BT


def _gelu(v):
    return v * 0.5 * (1.0 + jax.lax.erf(v * 0.7071067811865476))


def _moe_kernel(x_ref, W1_ref, b1_ref, W2_ref, b2_ref, W3_ref, b3_ref,
                Wg1_ref, bg1_ref, Wg2_ref, bg2_ref, temp_ref,
                out_ref, wdense_ref, acc_ref):
    e = pl.program_id(0)
    tb = pl.program_id(1)
    rows = pl.ds(tb * BT, BT)
    xb = x_ref[...]

    @pl.when(e == 0)
    def _gate():
        g = _gelu(jnp.dot(xb, Wg1_ref[...],
                          preferred_element_type=jnp.float32) + bg1_ref[...])
        gl = jnp.dot(g, Wg2_ref[...],
                     preferred_element_type=jnp.float32) + bg2_ref[...]
        iota = jax.lax.broadcasted_iota(jnp.int32, gl.shape, 1)
        m1 = jnp.max(gl, axis=-1, keepdims=True)
        i1 = jnp.min(jnp.where(gl == m1, iota, E), axis=-1, keepdims=True)
        oh1 = (iota == i1)
        masked = jnp.where(oh1, -jnp.inf, gl)
        m2 = jnp.max(masked, axis=-1, keepdims=True)
        i2 = jnp.min(jnp.where(masked == m2, iota, E), axis=-1, keepdims=True)
        oh2 = (iota == i2)
        e2 = jnp.exp(m2 - m1)
        w1 = 1.0 / (1.0 + e2)
        w2 = e2 / (1.0 + e2)
        wdense_ref[rows, :] = jnp.where(oh1, w1, 0.0) + jnp.where(oh2, w2, 0.0)

    h1 = _gelu(jnp.dot(xb, W1_ref[0],
                       preferred_element_type=jnp.float32) + b1_ref[0])
    h2 = _gelu(jnp.dot(h1, W2_ref[0],
                       preferred_element_type=jnp.float32) + b2_ref[0])
    o3 = jnp.dot(h2, W3_ref[0], preferred_element_type=jnp.float32) + b3_ref[0]

    wblk = wdense_ref[rows, :]
    sel = (jax.lax.broadcasted_iota(jnp.int32, (1, E), 1) == e)
    wv = jnp.sum(jnp.where(sel, wblk, 0.0), axis=-1, keepdims=True)
    contrib = wv * o3

    prev = acc_ref[rows, :]
    total = jnp.where(e == 0, contrib, prev + contrib)
    acc_ref[rows, :] = total

    @pl.when(e == E - 1)
    def _finish():
        t = jnp.maximum(temp_ref[0, 0], 1e-6)
        out_ref[rows, :] = total / t


@jax.jit
def kernel(x, W1, b1, W2, b2, W3, b3, Wg1, bg1, Wg2, bg2, temperature):
    bg1 = bg1.reshape(1, GATE_H)
    bg2 = bg2.reshape(1, E)
    b1 = b1.reshape(E, 1, HID)
    b2 = b2.reshape(E, 1, HID // 2)
    b3 = b3.reshape(E, 1, NC)
    temp = temperature.reshape(1, 1)

    grid = (E, TB)
    out = pl.pallas_call(
        _moe_kernel,
        grid=grid,
        in_specs=[
            pl.BlockSpec((BT, IN_DIM), lambda e, tb: (tb, 0)),
            pl.BlockSpec((1, IN_DIM, HID), lambda e, tb: (e, 0, 0)),
            pl.BlockSpec((1, 1, HID), lambda e, tb: (e, 0, 0)),
            pl.BlockSpec((1, HID, HID // 2), lambda e, tb: (e, 0, 0)),
            pl.BlockSpec((1, 1, HID // 2), lambda e, tb: (e, 0, 0)),
            pl.BlockSpec((1, HID // 2, NC), lambda e, tb: (e, 0, 0)),
            pl.BlockSpec((1, 1, NC), lambda e, tb: (e, 0, 0)),
            pl.BlockSpec((IN_DIM, GATE_H), lambda e, tb: (0, 0)),
            pl.BlockSpec((1, GATE_H), lambda e, tb: (0, 0)),
            pl.BlockSpec((GATE_H, E), lambda e, tb: (0, 0)),
            pl.BlockSpec((1, E), lambda e, tb: (0, 0)),
            pl.BlockSpec(memory_space=pltpu.SMEM),
        ],
        out_specs=pl.BlockSpec((TOKENS, NC), lambda e, tb: (0, 0)),
        out_shape=jax.ShapeDtypeStruct((TOKENS, NC), jnp.float32),
        scratch_shapes=[
            pltpu.VMEM((TOKENS, E), jnp.float32),
            pltpu.VMEM((TOKENS, NC), jnp.float32),
        ],
    )(x, W1, b1, W2, b2, W3, b3, Wg1, bg1, Wg2, bg2, temp)
    return out
